# baseline (speedup 1.0000x reference)
import jax
import jax.numpy as jnp
from jax import lax
from jax.experimental import pallas as pl
from jax.experimental.pallas import tpu as pltpu
from jax.experimental.pallas import tpu_sc as plsc

_ROWS = 1536
_COLS = 512
_PER_B = _ROWS * _COLS
_B_PER_BLOCK = 2
_NC = 2
_NW = 32
_NTAB = 1000

_B_SC = 16
_CH = 16384


def _sc_dense_body(x_hbm, n_hbm, t_hbm, sac_hbm, somac_hbm, o_hbm,
                   t_v, sac_v, somac_v, xb, nb, ob):
    w = lax.axis_index("s") * _NC + lax.axis_index("c")
    per_w = _B_SC * _PER_B // _NW
    nch = per_w // _CH

    pltpu.sync_copy(t_hbm, t_v)
    pltpu.sync_copy(sac_hbm, sac_v.at[pl.ds(0, _NTAB)])
    pltpu.sync_copy(somac_hbm, somac_v.at[pl.ds(0, _NTAB)])

    b_local = w // 2
    half = w % 2
    b_global = (x_hbm.shape[0] // _PER_B) - _B_SC + b_local
    tb = plsc.load_gather(t_v, [jnp.full((16,), b_global, jnp.int32)])
    a = plsc.load_gather(sac_v, [tb])
    s = plsc.load_gather(somac_v, [tb])

    xbase = b_global * _PER_B + half * per_w
    obase = b_local * _PER_B + half * per_w

    for i in range(nch):
        off = xbase + i * _CH
        pltpu.sync_copy(x_hbm.at[pl.ds(off, _CH)], xb)
        pltpu.sync_copy(n_hbm.at[pl.ds(off, _CH)], nb)

        def body(j, carry):
            for u in range(8):
                sl = pl.ds((j * 8 + u) * 16, 16)
                ob[sl] = a * xb[sl] + s * nb[sl]
            return carry

        lax.fori_loop(0, _CH // 128, body, 0)
        pltpu.sync_copy(ob, o_hbm.at[pl.ds(obase + i * _CH, _CH)])


def _sc_dense(x_flat_hi, n_flat_hi, t_hi, sac, somac):
    mesh = plsc.VectorSubcoreMesh(core_axis_name="c", subcore_axis_name="s")
    f = pl.kernel(
        _sc_dense_body,
        out_type=jax.ShapeDtypeStruct((_B_SC * _PER_B,), jnp.float32),
        mesh=mesh,
        compiler_params=pltpu.CompilerParams(needs_layout_passes=False),
        scratch_types=[
            pltpu.VMEM((64,), jnp.int32),
            pltpu.VMEM((1024,), jnp.float32),
            pltpu.VMEM((1024,), jnp.float32),
            pltpu.VMEM((_CH,), jnp.float32),
            pltpu.VMEM((_CH,), jnp.float32),
            pltpu.VMEM((_CH,), jnp.float32),
        ],
    )
    return f(x_flat_hi, n_flat_hi, t_hi, sac, somac)


def _fma_lo_body(t_ref, sac_ref, somac_ref, x_ref, n_ref, o_ref):
    bb = pl.program_id(0)
    for k in range(_B_PER_BLOCK):
        tt = t_ref[bb * _B_PER_BLOCK + k]
        o_ref[k] = sac_ref[tt] * x_ref[k] + somac_ref[tt] * n_ref[k]


def _merge_body(src_ref, io_ref, o_ref):
    o_ref[...] = src_ref[...]


def kernel(x_start, t, noise, sqrt_alphas_cumprod, sqrt_one_minus_alphas_cumprod):
    B, C, H, W = x_start.shape
    xr = x_start.reshape(B, _ROWS, _COLS)
    nr = noise.reshape(B, _ROWS, _COLS)
    t32 = t.astype(jnp.int32)
    b_tc = B - _B_SC

    blk = (_B_PER_BLOCK, _ROWS, _COLS)
    out_sds = jax.ShapeDtypeStruct((B, _ROWS, _COLS), jnp.float32)

    lo_spec = pl.BlockSpec(blk, lambda b, *_: (b, 0, 0))
    out1 = pl.pallas_call(
        _fma_lo_body,
        grid_spec=pltpu.PrefetchScalarGridSpec(
            num_scalar_prefetch=3,
            grid=(b_tc // _B_PER_BLOCK,),
            in_specs=[lo_spec, lo_spec],
            out_specs=lo_spec,
        ),
        out_shape=out_sds,
    )(t32, sqrt_alphas_cumprod, sqrt_one_minus_alphas_cumprod, xr, nr)

    sc_out = _sc_dense(x_start.reshape(B * _PER_B), noise.reshape(B * _PER_B),
                       t32, sqrt_alphas_cumprod, sqrt_one_minus_alphas_cumprod)

    nblk = b_tc // _B_PER_BLOCK
    src_spec = pl.BlockSpec(blk, lambda b: (b, 0, 0))
    dst_spec = pl.BlockSpec(blk, lambda b: (b + nblk, 0, 0))
    hbm_spec = pl.BlockSpec(memory_space=pltpu.MemorySpace.HBM)
    out = pl.pallas_call(
        _merge_body,
        grid=(_B_SC // _B_PER_BLOCK,),
        in_specs=[src_spec, hbm_spec],
        out_specs=dst_spec,
        out_shape=out_sds,
        input_output_aliases={1: 0},
    )(sc_out.reshape(_B_SC, _ROWS, _COLS), out1)
    return out.reshape(B, C, H, W)


# speedup vs baseline: 1.0047x; 1.0047x over previous
import jax
import jax.numpy as jnp
from jax import lax
from jax.experimental import pallas as pl
from jax.experimental.pallas import tpu as pltpu
from jax.experimental.pallas import tpu_sc as plsc

_ROWS = 1536
_COLS = 512
_PER_B = _ROWS * _COLS
_B_PER_BLOCK = 2
_NC = 2
_NW = 32
_NTAB = 1000

_B_SC = 16
_CH = 16384


def _sc_dense_body(x_hbm, n_hbm, t_hbm, sac_hbm, somac_hbm, o_hbm,
                   t_v, sac_v, somac_v, xb, nb, ob,
                   sx0, sx1, sn0, sn1, so0, so1):
    w = lax.axis_index("s") * _NC + lax.axis_index("c")
    per_w = _B_SC * _PER_B // _NW
    nch = per_w // _CH

    pltpu.sync_copy(t_hbm, t_v)
    pltpu.sync_copy(sac_hbm, sac_v.at[pl.ds(0, _NTAB)])
    pltpu.sync_copy(somac_hbm, somac_v.at[pl.ds(0, _NTAB)])

    b_local = w // 2
    half = w % 2
    b_global = (x_hbm.shape[0] // _PER_B) - _B_SC + b_local
    tb = plsc.load_gather(t_v, [jnp.full((16,), b_global, jnp.int32)])
    a = plsc.load_gather(sac_v, [tb])
    s = plsc.load_gather(somac_v, [tb])

    xbase = b_global * _PER_B + half * per_w
    obase = b_local * _PER_B + half * per_w

    sx = (sx0, sx1)
    sn = (sn0, sn1)
    so = (so0, so1)
    h_x = [None, None]
    h_n = [None, None]
    h_o = [None, None]

    def start_in(i, slot):
        off = xbase + i * _CH
        h_x[slot] = pltpu.async_copy(x_hbm.at[pl.ds(off, _CH)], xb.at[slot], sx[slot])
        h_n[slot] = pltpu.async_copy(n_hbm.at[pl.ds(off, _CH)], nb.at[slot], sn[slot])

    start_in(0, 0)
    for i in range(nch):
        slot = i & 1
        if i + 1 < nch:
            start_in(i + 1, 1 - slot)
        h_x[slot].wait()
        h_n[slot].wait()
        if h_o[slot] is not None:
            h_o[slot].wait()

        def body(j, carry):
            for u in range(8):
                sl = pl.ds((j * 8 + u) * 16, 16)
                ob[slot, sl] = a * xb[slot, sl] + s * nb[slot, sl]
            return carry

        lax.fori_loop(0, _CH // 128, body, 0)
        h_o[slot] = pltpu.async_copy(
            ob.at[slot], o_hbm.at[pl.ds(obase + i * _CH, _CH)], so[slot])
    h_o[0].wait()
    h_o[1].wait()


def _sc_dense(x_flat_hi, n_flat_hi, t_hi, sac, somac):
    mesh = plsc.VectorSubcoreMesh(core_axis_name="c", subcore_axis_name="s")
    f = pl.kernel(
        _sc_dense_body,
        out_type=jax.ShapeDtypeStruct((_B_SC * _PER_B,), jnp.float32),
        mesh=mesh,
        compiler_params=pltpu.CompilerParams(needs_layout_passes=False),
        scratch_types=[
            pltpu.VMEM((64,), jnp.int32),
            pltpu.VMEM((1024,), jnp.float32),
            pltpu.VMEM((1024,), jnp.float32),
            pltpu.VMEM((2, _CH), jnp.float32),
            pltpu.VMEM((2, _CH), jnp.float32),
            pltpu.VMEM((2, _CH), jnp.float32),
            pltpu.SemaphoreType.DMA,
            pltpu.SemaphoreType.DMA,
            pltpu.SemaphoreType.DMA,
            pltpu.SemaphoreType.DMA,
            pltpu.SemaphoreType.DMA,
            pltpu.SemaphoreType.DMA,
        ],
    )
    return f(x_flat_hi, n_flat_hi, t_hi, sac, somac)


def _fma_lo_body(t_ref, sac_ref, somac_ref, x_ref, n_ref, o_ref):
    bb = pl.program_id(0)
    for k in range(_B_PER_BLOCK):
        tt = t_ref[bb * _B_PER_BLOCK + k]
        o_ref[k] = sac_ref[tt] * x_ref[k] + somac_ref[tt] * n_ref[k]


def _merge_body(src_ref, io_ref, o_ref):
    o_ref[...] = src_ref[...]


def kernel(x_start, t, noise, sqrt_alphas_cumprod, sqrt_one_minus_alphas_cumprod):
    B, C, H, W = x_start.shape
    xr = x_start.reshape(B, _ROWS, _COLS)
    nr = noise.reshape(B, _ROWS, _COLS)
    t32 = t.astype(jnp.int32)
    b_tc = B - _B_SC

    blk = (_B_PER_BLOCK, _ROWS, _COLS)
    out_sds = jax.ShapeDtypeStruct((B, _ROWS, _COLS), jnp.float32)

    lo_spec = pl.BlockSpec(blk, lambda b, *_: (b, 0, 0))
    out1 = pl.pallas_call(
        _fma_lo_body,
        grid_spec=pltpu.PrefetchScalarGridSpec(
            num_scalar_prefetch=3,
            grid=(b_tc // _B_PER_BLOCK,),
            in_specs=[lo_spec, lo_spec],
            out_specs=lo_spec,
        ),
        out_shape=out_sds,
    )(t32, sqrt_alphas_cumprod, sqrt_one_minus_alphas_cumprod, xr, nr)

    sc_out = _sc_dense(x_start.reshape(B * _PER_B), noise.reshape(B * _PER_B),
                       t32, sqrt_alphas_cumprod, sqrt_one_minus_alphas_cumprod)

    nblk = b_tc // _B_PER_BLOCK
    src_spec = pl.BlockSpec(blk, lambda b: (b, 0, 0))
    dst_spec = pl.BlockSpec(blk, lambda b: (b + nblk, 0, 0))
    hbm_spec = pl.BlockSpec(memory_space=pltpu.MemorySpace.HBM)
    out = pl.pallas_call(
        _merge_body,
        grid=(_B_SC // _B_PER_BLOCK,),
        in_specs=[src_spec, hbm_spec],
        out_specs=dst_spec,
        out_shape=out_sds,
        input_output_aliases={1: 0},
    )(sc_out.reshape(_B_SC, _ROWS, _COLS), out1)
    return out.reshape(B, C, H, W)


# speedup vs baseline: 2.8636x; 2.8501x over previous
import jax
import jax.numpy as jnp
from jax import lax
from jax.experimental import pallas as pl
from jax.experimental.pallas import tpu as pltpu
from jax.experimental.pallas import tpu_sc as plsc

_ROWS = 1536
_COLS = 512
_B_PER_BLOCK = 2
_NC = 1
_NTAB = 1000


def _sc_gather_body(t_hbm, sac_hbm, somac_hbm, a_hbm, s_hbm,
                    t_v, sac_v, somac_v, a_v, s_v):
    wid = lax.axis_index("s") * _NC + lax.axis_index("c")

    @pl.when(wid < 4)
    def _():
        base = wid * 16
        pltpu.sync_copy(t_hbm.at[pl.ds(base, 16)], t_v)
        pltpu.sync_copy(sac_hbm, sac_v.at[pl.ds(0, _NTAB)])
        pltpu.sync_copy(somac_hbm, somac_v.at[pl.ds(0, _NTAB)])
        idx = t_v[...]
        a_v[...] = plsc.load_gather(sac_v, [idx])
        s_v[...] = plsc.load_gather(somac_v, [idx])
        pltpu.sync_copy(a_v, a_hbm.at[pl.ds(base, 16)])
        pltpu.sync_copy(s_v, s_hbm.at[pl.ds(base, 16)])


def _sc_gather(t32, sac, somac):
    B = t32.shape[0]
    mesh = plsc.VectorSubcoreMesh(core_axis_name="c", subcore_axis_name="s",
                                  num_cores=_NC)
    f = pl.kernel(
        _sc_gather_body,
        out_type=(
            jax.ShapeDtypeStruct((B,), jnp.float32),
            jax.ShapeDtypeStruct((B,), jnp.float32),
        ),
        mesh=mesh,
        compiler_params=pltpu.CompilerParams(
            needs_layout_passes=False,
            skip_device_barrier=True,
        ),
        scratch_types=[
            pltpu.VMEM((16,), jnp.int32),
            pltpu.VMEM((1024,), jnp.float32),
            pltpu.VMEM((1024,), jnp.float32),
            pltpu.VMEM((16,), jnp.float32),
            pltpu.VMEM((16,), jnp.float32),
        ],
    )
    return f(t32, sac, somac)


def _fma_lo_body(t_ref, sac_ref, somac_ref, x_ref, n_ref, o_ref):
    bb = pl.program_id(0)
    for k in range(_B_PER_BLOCK):
        tt = t_ref[bb * _B_PER_BLOCK + k]
        o_ref[k] = sac_ref[tt] * x_ref[k] + somac_ref[tt] * n_ref[k]


def _fma_hi_body(a_ref, s_ref, x_ref, n_ref, io_ref, o_ref):
    bb = pl.program_id(0)
    half = pl.num_programs(0) * _B_PER_BLOCK
    for k in range(_B_PER_BLOCK):
        b = half + bb * _B_PER_BLOCK + k
        o_ref[k] = a_ref[b] * x_ref[k] + s_ref[b] * n_ref[k]


def kernel(x_start, t, noise, sqrt_alphas_cumprod, sqrt_one_minus_alphas_cumprod):
    B, C, H, W = x_start.shape
    xr = x_start.reshape(B, _ROWS, _COLS)
    nr = noise.reshape(B, _ROWS, _COLS)
    t32 = t.astype(jnp.int32)
    half = B // 2

    blk = (_B_PER_BLOCK, _ROWS, _COLS)
    grid = (half // _B_PER_BLOCK,)
    out_sds = jax.ShapeDtypeStruct((B, _ROWS, _COLS), jnp.float32)

    lo_spec = pl.BlockSpec(blk, lambda b, *_: (b, 0, 0))
    out1 = pl.pallas_call(
        _fma_lo_body,
        grid_spec=pltpu.PrefetchScalarGridSpec(
            num_scalar_prefetch=3,
            grid=grid,
            in_specs=[lo_spec, lo_spec],
            out_specs=lo_spec,
        ),
        out_shape=out_sds,
    )(t32, sqrt_alphas_cumprod, sqrt_one_minus_alphas_cumprod, xr, nr)

    a_vec, s_vec = _sc_gather(t32, sqrt_alphas_cumprod, sqrt_one_minus_alphas_cumprod)

    nblk = half // _B_PER_BLOCK
    hi_spec = pl.BlockSpec(blk, lambda b, *_: (b + nblk, 0, 0))
    hbm_spec = pl.BlockSpec(memory_space=pltpu.MemorySpace.HBM)
    out = pl.pallas_call(
        _fma_hi_body,
        grid_spec=pltpu.PrefetchScalarGridSpec(
            num_scalar_prefetch=2,
            grid=grid,
            in_specs=[hi_spec, hi_spec, hbm_spec],
            out_specs=hi_spec,
        ),
        out_shape=out_sds,
        input_output_aliases={4: 0},
    )(a_vec, s_vec, xr, nr, out1)
    return out.reshape(B, C, H, W)


# speedup vs baseline: 2.8652x; 1.0006x over previous
import jax
import jax.numpy as jnp
from jax import lax
from jax.experimental import pallas as pl
from jax.experimental.pallas import tpu as pltpu
from jax.experimental.pallas import tpu_sc as plsc

_ROWS = 1536
_COLS = 512
_B_PER_BLOCK = 2
_NC = 1
_NTAB = 1000


def _sc_gather_body(t_hbm, sac_hbm, somac_hbm, a_hbm, s_hbm,
                    t_v, sac_v, somac_v, a_v, s_v):
    wid = lax.axis_index("s") * _NC + lax.axis_index("c")

    @pl.when(wid < 4)
    def _():
        base = wid * 16
        pltpu.sync_copy(t_hbm.at[pl.ds(base, 16)], t_v)
        pltpu.sync_copy(sac_hbm, sac_v.at[pl.ds(0, _NTAB)])
        pltpu.sync_copy(somac_hbm, somac_v.at[pl.ds(0, _NTAB)])
        idx = t_v[...]
        a_v[...] = plsc.load_gather(sac_v, [idx])
        s_v[...] = plsc.load_gather(somac_v, [idx])
        pltpu.sync_copy(a_v, a_hbm.at[pl.ds(base, 16)])
        pltpu.sync_copy(s_v, s_hbm.at[pl.ds(base, 16)])


def _sc_gather(t32, sac, somac):
    B = t32.shape[0]
    mesh = plsc.VectorSubcoreMesh(core_axis_name="c", subcore_axis_name="s",
                                  num_cores=_NC)
    f = pl.kernel(
        _sc_gather_body,
        out_type=(
            jax.ShapeDtypeStruct((B,), jnp.float32),
            jax.ShapeDtypeStruct((B,), jnp.float32),
        ),
        mesh=mesh,
        compiler_params=pltpu.CompilerParams(needs_layout_passes=False),
        scratch_types=[
            pltpu.VMEM((16,), jnp.int32),
            pltpu.VMEM((1024,), jnp.float32),
            pltpu.VMEM((1024,), jnp.float32),
            pltpu.VMEM((16,), jnp.float32),
            pltpu.VMEM((16,), jnp.float32),
        ],
    )
    return f(t32, sac, somac)


def _fma_lo_body(t_ref, sac_ref, somac_ref, x_ref, n_ref, o_ref):
    bb = pl.program_id(0)
    for k in range(_B_PER_BLOCK):
        tt = t_ref[bb * _B_PER_BLOCK + k]
        o_ref[k] = sac_ref[tt] * x_ref[k] + somac_ref[tt] * n_ref[k]


def _fma_hi_body(a_ref, s_ref, x_ref, n_ref, io_ref, o_ref):
    bb = pl.program_id(0)
    half = pl.num_programs(0) * _B_PER_BLOCK
    for k in range(_B_PER_BLOCK):
        b = half + bb * _B_PER_BLOCK + k
        o_ref[k] = a_ref[b] * x_ref[k] + s_ref[b] * n_ref[k]


def kernel(x_start, t, noise, sqrt_alphas_cumprod, sqrt_one_minus_alphas_cumprod):
    B, C, H, W = x_start.shape
    xr = x_start.reshape(B, _ROWS, _COLS)
    nr = noise.reshape(B, _ROWS, _COLS)
    t32 = t.astype(jnp.int32)
    half = B // 2

    blk = (_B_PER_BLOCK, _ROWS, _COLS)
    grid = (half // _B_PER_BLOCK,)
    out_sds = jax.ShapeDtypeStruct((B, _ROWS, _COLS), jnp.float32)

    lo_spec = pl.BlockSpec(blk, lambda b, *_: (b, 0, 0))
    out1 = pl.pallas_call(
        _fma_lo_body,
        grid_spec=pltpu.PrefetchScalarGridSpec(
            num_scalar_prefetch=3,
            grid=grid,
            in_specs=[lo_spec, lo_spec],
            out_specs=lo_spec,
        ),
        out_shape=out_sds,
    )(t32, sqrt_alphas_cumprod, sqrt_one_minus_alphas_cumprod, xr, nr)

    a_vec, s_vec = _sc_gather(t32, sqrt_alphas_cumprod, sqrt_one_minus_alphas_cumprod)

    nblk = half // _B_PER_BLOCK
    hi_spec = pl.BlockSpec(blk, lambda b, *_: (b + nblk, 0, 0))
    hbm_spec = pl.BlockSpec(memory_space=pltpu.MemorySpace.HBM)
    out = pl.pallas_call(
        _fma_hi_body,
        grid_spec=pltpu.PrefetchScalarGridSpec(
            num_scalar_prefetch=2,
            grid=grid,
            in_specs=[hi_spec, hi_spec, hbm_spec],
            out_specs=hi_spec,
        ),
        out_shape=out_sds,
        input_output_aliases={4: 0},
    )(a_vec, s_vec, xr, nr, out1)
    return out.reshape(B, C, H, W)
